# Initial kernel scaffold; baseline (speedup 1.0000x reference)
import jax
import jax.numpy as jnp
from jax.experimental import pallas as pl


def kernel(hidden_states, ln1_weight, ln1_bias, ln2_weight, ln2_bias, qkv_weight, proj_weight, router_weight, moe_w1, moe_w2):
    raise NotImplementedError("write your pallas kernel here")


# speedup vs baseline: 1.1142x; 1.1142x over previous
import functools

import jax
import jax.numpy as jnp
from jax.experimental import pallas as pl
from jax.experimental.pallas import tpu as pltpu

S, H = 2048, 1024
NH, HD = 16, 64
E, TOPK, DFF = 8, 2, 1024
LN_EPS = 1e-5

QT = 256
KT = 256
RT = 256


def _ln(x, w, b):
    mu = jnp.mean(x, axis=-1, keepdims=True)
    var = jnp.mean((x - mu) ** 2, axis=-1, keepdims=True)
    return (x - mu) * jax.lax.rsqrt(var + LN_EPS) * w + b


def _k1_body(x_ref, lw_ref, lb_ref, w_ref, qkv_ref):
    x = x_ref[...]
    ln = _ln(x, lw_ref[...], lb_ref[...])
    qkv_ref[...] = jnp.dot(ln.astype(jnp.bfloat16), w_ref[...].T,
                           preferred_element_type=jnp.float32).astype(jnp.bfloat16)


def _k1(x, lw, lb, wbf):
    return pl.pallas_call(
        _k1_body,
        grid=(S // RT,),
        in_specs=[
            pl.BlockSpec((RT, H), lambda i: (i, 0)),
            pl.BlockSpec((H,), lambda i: (0,)),
            pl.BlockSpec((H,), lambda i: (0,)),
            pl.BlockSpec((3 * H, H), lambda i: (0, 0)),
        ],
        out_specs=pl.BlockSpec((RT, 3 * H), lambda i: (i, 0)),
        out_shape=jax.ShapeDtypeStruct((S, 3 * H), jnp.bfloat16),
    )(x, lw, lb, wbf)


def _k2_body(q_ref, k_ref, v_ref, o_ref):
    i = pl.program_id(1)
    scale = 1.0 / (HD ** 0.5)
    rows = jax.lax.broadcasted_iota(jnp.int32, (QT, KT), 0) + i * QT

    for hh in range(2):
        sl = slice(hh * HD, (hh + 1) * HD)
        q = (q_ref[:, sl].astype(jnp.float32) * scale).astype(jnp.bfloat16)
        m0 = jnp.full((QT, 1), -1e30, jnp.float32)
        l0 = jnp.zeros((QT, 1), jnp.float32)
        a0 = jnp.zeros((QT, HD), jnp.float32)

        def step(j, carry):
            m, l, acc = carry
            kj = k_ref[pl.ds(j * KT, KT), sl]
            vj = v_ref[pl.ds(j * KT, KT), sl]
            s = jax.lax.dot_general(q, kj, (((1,), (1,)), ((), ())),
                                    preferred_element_type=jnp.float32)
            cols = jax.lax.broadcasted_iota(jnp.int32, (QT, KT), 1) + j * KT
            s = jnp.where(rows >= cols, s, -1e30)
            mj = jnp.maximum(m, jnp.max(s, axis=1, keepdims=True))
            p = jnp.exp(s - mj)
            corr = jnp.exp(m - mj)
            l = l * corr + jnp.sum(p, axis=1, keepdims=True)
            acc = acc * corr + jnp.dot(p.astype(jnp.bfloat16), vj,
                                       preferred_element_type=jnp.float32)
            return mj, l, acc

        _, l, acc = jax.lax.fori_loop(0, i + 1, step, (m0, l0, a0))
        o_ref[:, sl] = (acc / l).astype(jnp.bfloat16)


def _k2(qkv):
    return pl.pallas_call(
        _k2_body,
        grid=(NH // 2, S // QT),
        in_specs=[
            pl.BlockSpec((QT, 2 * HD), lambda h, i: (i, h)),
            pl.BlockSpec((S, 2 * HD), lambda h, i: (0, 8 + h)),
            pl.BlockSpec((S, 2 * HD), lambda h, i: (0, 16 + h)),
        ],
        out_specs=pl.BlockSpec((QT, 2 * HD), lambda h, i: (i, h)),
        out_shape=jax.ShapeDtypeStruct((S, H), jnp.bfloat16),
    )(qkv, qkv, qkv)


def _k3_body(a_ref, x_ref, pw_ref, lw_ref, lb_ref, rw_ref,
             h2_ref, flat_ref, gate_ref):
    proj = jnp.dot(a_ref[...], pw_ref[...].T, preferred_element_type=jnp.float32)
    h2 = x_ref[...] + proj
    h2_ref[...] = h2
    flat = _ln(h2, lw_ref[...], lb_ref[...])
    flat_ref[...] = flat
    logits = jax.lax.dot_general(
        flat, rw_ref[...], (((1,), (1,)), ((), ())),
        preferred_element_type=jnp.float32,
        precision=jax.lax.Precision.HIGHEST)
    m = jnp.max(logits, axis=1, keepdims=True)
    ex = jnp.exp(logits - m)
    p = ex / jnp.sum(ex, axis=1, keepdims=True)
    m1 = jnp.max(logits, axis=1, keepdims=True)
    l2 = jnp.where(logits == m1, -jnp.inf, logits)
    m2 = jnp.max(l2, axis=1, keepdims=True)
    mask = logits >= m2
    gate_ref[...] = jnp.where(mask, p, 0.0)


def _k3(attn, x, pwbf, lw, lb, rw):
    return pl.pallas_call(
        _k3_body,
        grid=(S // RT,),
        in_specs=[
            pl.BlockSpec((RT, H), lambda i: (i, 0)),
            pl.BlockSpec((RT, H), lambda i: (i, 0)),
            pl.BlockSpec((H, H), lambda i: (0, 0)),
            pl.BlockSpec((H,), lambda i: (0,)),
            pl.BlockSpec((H,), lambda i: (0,)),
            pl.BlockSpec((E, H), lambda i: (0, 0)),
        ],
        out_specs=[
            pl.BlockSpec((RT, H), lambda i: (i, 0)),
            pl.BlockSpec((RT, H), lambda i: (i, 0)),
            pl.BlockSpec((RT, E), lambda i: (i, 0)),
        ],
        out_shape=[
            jax.ShapeDtypeStruct((S, H), jnp.float32),
            jax.ShapeDtypeStruct((S, H), jnp.float32),
            jax.ShapeDtypeStruct((S, E), jnp.float32),
        ],
    )(attn, x, pwbf, lw, lb, rw)


def _k4_body(flat_ref, gate_ref, h2_ref, w1_ref, w2_ref, out_ref):
    e = pl.program_id(0)
    onehot = (jax.lax.broadcasted_iota(jnp.int32, (E, 1), 0) == e
              ).astype(jnp.float32)
    g = jnp.dot(gate_ref[...], onehot, preferred_element_type=jnp.float32)
    x = flat_ref[...].astype(jnp.bfloat16)
    h = jax.lax.dot_general(x, w1_ref[0], (((1,), (1,)), ((), ())),
                            preferred_element_type=jnp.float32)
    h = h * jax.nn.sigmoid(h) * g
    y = jax.lax.dot_general(h.astype(jnp.bfloat16), w2_ref[0],
                            (((1,), (1,)), ((), ())),
                            preferred_element_type=jnp.float32)

    @pl.when(e == 0)
    def _():
        out_ref[...] = h2_ref[...] + y

    @pl.when(e > 0)
    def _():
        out_ref[...] += y


def _k4(flat, gate, h2, w1bf, w2bf):
    return pl.pallas_call(
        _k4_body,
        grid=(E,),
        in_specs=[
            pl.BlockSpec((S, H), lambda e: (0, 0)),
            pl.BlockSpec((S, E), lambda e: (0, 0)),
            pl.BlockSpec((S, H), lambda e: (0, 0)),
            pl.BlockSpec((1, DFF, H), lambda e: (e, 0, 0)),
            pl.BlockSpec((1, H, DFF), lambda e: (e, 0, 0)),
        ],
        out_specs=pl.BlockSpec((S, H), lambda e: (0, 0)),
        out_shape=jax.ShapeDtypeStruct((S, H), jnp.float32),
    )(flat, gate, h2, w1bf, w2bf)


def kernel(hidden_states, ln1_weight, ln1_bias, ln2_weight, ln2_bias,
           qkv_weight, proj_weight, router_weight, moe_w1, moe_w2):
    x = hidden_states.reshape(S, H)
    qkv = _k1(x, ln1_weight, ln1_bias, qkv_weight.astype(jnp.bfloat16))
    attn = _k2(qkv)
    h2, flat, gate = _k3(attn, x, proj_weight.astype(jnp.bfloat16),
                         ln2_weight, ln2_bias, router_weight)
    out = _k4(flat, gate, h2, moe_w1.astype(jnp.bfloat16),
              moe_w2.astype(jnp.bfloat16))
    return out.reshape(S, 1, H)
